# Initial kernel scaffold; baseline (speedup 1.0000x reference)
import jax
import jax.numpy as jnp
from jax.experimental import pallas as pl


def kernel(label, feat, centers):
    raise NotImplementedError("write your pallas kernel here")


# speedup vs baseline: 1.0732x; 1.0732x over previous
import functools

import jax
import jax.numpy as jnp
from jax import lax
from jax.experimental import pallas as pl
from jax.experimental.pallas import tpu as pltpu
from jax.experimental.pallas import tpu_sc as plsc

NUM_CLASSES = 100000
FEAT_DIM = 128
BATCH = 4096

_NC = 2
_NS = 16
_NW = _NC * _NS
_L = 16
_BPW = BATCH // _NW
_CHUNKS = FEAT_DIM // _L


def _sc_partials(label, feat, centers):
    mesh = plsc.VectorSubcoreMesh(core_axis_name="c", subcore_axis_name="s")

    @functools.partial(
        pl.kernel,
        out_type=jax.ShapeDtypeStruct((_NW, _L), jnp.float32),
        mesh=mesh,
        scratch_types=[
            pltpu.VMEM((_BPW,), jnp.int32),
            pltpu.VMEM((_BPW, FEAT_DIM), jnp.float32),
            pltpu.VMEM((_BPW, FEAT_DIM), jnp.float32),
            pltpu.VMEM((_L,), jnp.float32),
            pltpu.SemaphoreType.DMA,
        ],
    )
    def k(label_hbm, feat_hbm, centers_hbm, out_hbm, idx_v, rows_v, feat_v,
          acc_v, sem):
        wid = lax.axis_index("s") * _NC + lax.axis_index("c")
        base = wid * _BPW

        pltpu.sync_copy(label_hbm.at[pl.ds(base, _BPW)], idx_v)
        gather = pltpu.async_copy(centers_hbm.at[idx_v], rows_v, sem)
        pltpu.sync_copy(feat_hbm.at[pl.ds(base, _BPW), :], feat_v)
        gather.wait()

        def row_body(r, acc):
            for c in range(_CHUNKS):
                d = feat_v[r, pl.ds(c * _L, _L)] - rows_v[r, pl.ds(c * _L, _L)]
                acc = acc + d * d
            return acc

        acc = lax.fori_loop(0, _BPW, row_body,
                            jnp.zeros((_L,), jnp.float32))
        acc_v[...] = acc
        pltpu.sync_copy(acc_v, out_hbm.at[wid])

    return k(label, feat, centers)


def _tc_reduce(partials):
    def red(x_ref, o_ref):
        o_ref[...] = (jnp.sum(x_ref[...]) * 0.5).reshape(1, 1)

    return pl.pallas_call(
        red,
        out_shape=jax.ShapeDtypeStruct((1, 1), jnp.float32),
    )(partials)


@jax.jit
def kernel(label, feat, centers):
    label = label.astype(jnp.int32)
    partials = _sc_partials(label, feat, centers)
    return _tc_reduce(partials.reshape(4, 128))[0, 0]


# speedup vs baseline: 1.1091x; 1.0335x over previous
import functools

import jax
import jax.numpy as jnp
from jax import lax
from jax.experimental import pallas as pl
from jax.experimental.pallas import tpu as pltpu
from jax.experimental.pallas import tpu_sc as plsc

NUM_CLASSES = 100000
FEAT_DIM = 128
BATCH = 4096

_NC = 2
_NS = 16
_NW = _NC * _NS
_L = 16
_BPW = BATCH // _NW
_CHUNKS = FEAT_DIM // _L
_GCH = 4
_RPC = _BPW // _GCH


def _sc_partials(label, feat, centers):
    mesh = plsc.VectorSubcoreMesh(core_axis_name="c", subcore_axis_name="s")

    @functools.partial(
        pl.kernel,
        out_type=jax.ShapeDtypeStruct((_NW // 8, 8 * _L), jnp.float32),
        mesh=mesh,
        scratch_types=[
            pltpu.VMEM((_BPW,), jnp.int32),
            pltpu.VMEM((_BPW, FEAT_DIM), jnp.float32),
            pltpu.VMEM((_BPW, FEAT_DIM), jnp.float32),
            pltpu.VMEM((_L,), jnp.float32),
            pltpu.SemaphoreType.DMA,
        ] + [pltpu.SemaphoreType.DMA] * _GCH,
    )
    def k(label_hbm, feat_hbm, centers_hbm, out_hbm, idx_v, rows_v, feat_v,
          acc_v, fsem, *gsems):
        wid = lax.axis_index("s") * _NC + lax.axis_index("c")
        base = wid * _BPW

        pltpu.sync_copy(label_hbm.at[pl.ds(base, _BPW)], idx_v)
        fcopy = pltpu.async_copy(feat_hbm.at[pl.ds(base, _BPW), :], feat_v,
                                 fsem)
        gathers = []
        for g in range(_GCH):
            gathers.append(pltpu.async_copy(
                centers_hbm.at[idx_v.at[pl.ds(g * _RPC, _RPC)]],
                rows_v.at[pl.ds(g * _RPC, _RPC), :],
                gsems[g]))
        fcopy.wait()

        zero = jnp.zeros((_L,), jnp.float32)

        def row_body(r, accs):
            out = []
            for c in range(_CHUNKS):
                d = feat_v[r, pl.ds(c * _L, _L)] - rows_v[r, pl.ds(c * _L, _L)]
                out.append(accs[c] + d * d)
            return tuple(out)

        accs = (zero,) * _CHUNKS
        for g in range(_GCH):
            gathers[g].wait()
            accs = lax.fori_loop(g * _RPC, (g + 1) * _RPC, row_body, accs)

        acc = accs[0]
        for c in range(1, _CHUNKS):
            acc = acc + accs[c]
        acc_v[...] = acc
        pltpu.sync_copy(acc_v, out_hbm.at[wid // 8, pl.ds((wid % 8) * _L, _L)])

    return k(label, feat, centers)


def _tc_reduce(partials):
    def red(x_ref, o_ref):
        o_ref[...] = (jnp.sum(x_ref[...]) * 0.5).reshape(1, 1)

    return pl.pallas_call(
        red,
        out_shape=jax.ShapeDtypeStruct((1, 1), jnp.float32),
    )(partials)


@jax.jit
def kernel(label, feat, centers):
    label = label.astype(jnp.int32)
    partials = _sc_partials(label, feat, centers)
    return _tc_reduce(partials).reshape(())
